# baseline (speedup 1.0000x reference)
import functools

import jax
import jax.numpy as jnp
from jax import lax
from jax.experimental import pallas as pl
from jax.experimental.pallas import tpu as pltpu
from jax.experimental.pallas import tpu_sc as plsc

_EPS = 1e-5
_G = 64
_NC = 2
_NS = 16
_LANE = 128


def _mesh():
  return plsc.VectorSubcoreMesh(
      core_axis_name="c", subcore_axis_name="s",
      num_cores=_NC, num_subcores=_NS)


_SC_PARAMS = pltpu.CompilerParams(use_tc_tiling_on_sc=False)


def _deg_kernel(kch, nacc):
  @functools.partial(
      pl.kernel,
      out_type=jax.ShapeDtypeStruct((_NC * nacc,), jnp.float32),
      mesh=_mesh(),
      compiler_params=_SC_PARAMS,
      scratch_types=[
          pltpu.VMEM((kch, _LANE), jnp.int32),
          pltpu.VMEM((_LANE,), jnp.float32),
          pltpu.VMEM_SHARED((nacc,), jnp.float32),
      ])
  def dk(dst_hbm, out_hbm, didx, vals, acc):
    c = lax.axis_index("c")
    s = lax.axis_index("s")
    wid = c * _NS + s

    @pl.loop(0, _LANE // 16)
    def _zero_vals(i):
      vals[pl.ds(i * 16, 16)] = jnp.zeros((16,), jnp.float32)

    base = s * (nacc // _NS)

    @pl.loop(0, nacc // _NS // _LANE)
    def _zero_acc(i):
      pltpu.sync_copy(vals, acc.at[pl.ds(base + i * _LANE, _LANE)])

    @pl.loop(0, _LANE // 16)
    def _one_vals(i):
      vals[pl.ds(i * 16, 16)] = jnp.ones((16,), jnp.float32)

    pltpu.sync_copy(dst_hbm.at[pl.ds(wid * kch, kch)], didx)
    plsc.subcore_barrier()

    @pl.loop(0, kch)
    def _edges(j):
      pltpu.sync_copy(vals, acc.at[didx.at[j]], add=True)

    plsc.subcore_barrier()
    rpt = nacc // _NS
    pltpu.sync_copy(acc.at[pl.ds(s * rpt, rpt)],
                    out_hbm.at[pl.ds(c * nacc + s * rpt, rpt)])

  return dk


def _scatter_kernel(kch, nacc, F):
  @functools.partial(
      pl.kernel,
      out_type=jax.ShapeDtypeStruct((_NC, nacc, F), jnp.float32),
      mesh=_mesh(),
      compiler_params=_SC_PARAMS,
      scratch_types=[
          pltpu.VMEM((kch, _LANE), jnp.int32),
          pltpu.VMEM((kch, _LANE), jnp.int32),
          pltpu.VMEM((2, _LANE, F), jnp.float32),
          pltpu.VMEM_SHARED((nacc, F), jnp.float32),
          pltpu.SemaphoreType.DMA,
          pltpu.SemaphoreType.DMA,
      ])
  def sk(src_hbm, dst_hbm, hp_hbm, out_hbm, sidx, didx, rows, acc, sem0,
         sem1):
    c = lax.axis_index("c")
    s = lax.axis_index("s")
    wid = (1 - c) * _NS + s
    sems = (sem0, sem1)

    @pl.loop(0, _LANE)
    def _zero_rows(i):
      for j in range(F // 16):
        rows[0, i, pl.ds(j * 16, 16)] = jnp.zeros((16,), jnp.float32)

    base = s * (nacc // _NS)

    @pl.loop(0, nacc // _NS // _LANE)
    def _zero_acc(i):
      pltpu.sync_copy(rows.at[0], acc.at[pl.ds(base + i * _LANE, _LANE)])

    pltpu.sync_copy(src_hbm.at[pl.ds(wid * kch, kch)], sidx)
    pltpu.sync_copy(dst_hbm.at[pl.ds(wid * kch, kch)], didx)
    plsc.subcore_barrier()

    def _fire(j, b):
      pltpu.async_copy(hp_hbm.at[sidx.at[j]], rows.at[b], sems[b])

    def _drain_scatter(j, b):
      pltpu.make_async_copy(hp_hbm.at[sidx.at[j]], rows.at[b],
                            sems[b]).wait()
      pltpu.sync_copy(rows.at[b], acc.at[didx.at[j]], add=True)

    _fire(0, 0)

    @pl.loop(0, kch // 2)
    def _edges(p):
      j = p * 2
      _fire(j + 1, 1)
      _drain_scatter(j, 0)

      @pl.when(j + 2 < kch)
      def _next():
        _fire(j + 2, 0)

      _drain_scatter(j + 1, 1)

    plsc.subcore_barrier()
    rpt = nacc // _NS
    ob = s * rpt
    pltpu.sync_copy(acc.at[pl.ds(ob, rpt)], out_hbm.at[c, pl.ds(ob, rpt)])

  return sk


def _proj1_body(x_ref, w_ref, deg_ref, out_ref):
  n = x_ref.shape[0]
  dinv = lax.rsqrt(deg_ref[0, :n] + deg_ref[1, :n] + 1.0)
  h = jnp.dot(x_ref[...], w_ref[...], preferred_element_type=jnp.float32)
  out_ref[...] = h * dinv


def _mid_body(p_ref, hp_ref, deg_ref, b_ref, g_ref, be_ref, w_ref, out_ref):
  n = hp_ref.shape[0]
  dinv = lax.rsqrt(deg_ref[0, :n] + deg_ref[1, :n] + 1.0)
  tmp = p_ref[0, :n] + p_ref[1, :n] + hp_ref[...]
  h = tmp * dinv + b_ref[...]
  t = jnp.tanh(h)
  mu = jnp.mean(t, axis=0, keepdims=True)
  var = jnp.mean((t - mu) * (t - mu), axis=0, keepdims=True)
  hbn = g_ref[...] * (t - mu) * lax.rsqrt(var + _EPS) + be_ref[...]
  out_ref[...] = jnp.dot(
      hbn, w_ref[...], preferred_element_type=jnp.float32) * dinv


def _final_body(p_ref, hp_ref, deg_ref, b_ref, batch_ref, wc_ref, bc_ref,
                out_ref):
  n = hp_ref.shape[0]
  dinv = lax.rsqrt(deg_ref[0, :n] + deg_ref[1, :n] + 1.0)
  h = (p_ref[0, :n] + p_ref[1, :n] + hp_ref[...]) * dinv + b_ref[...]
  t = jnp.tanh(h)
  gid = lax.broadcasted_iota(jnp.int32, (_G, n), 0)
  onehot = (gid == batch_ref[...]).astype(jnp.float32)
  sums = jnp.dot(onehot, t, preferred_element_type=jnp.float32)
  counts = jnp.sum(onehot, axis=1, keepdims=True)
  pooled = sums / jnp.maximum(counts, 1.0)
  out_ref[...] = jnp.dot(
      pooled, wc_ref[...], preferred_element_type=jnp.float32) + bc_ref[...]


def kernel(x, edge_index, batch, W1, b1, g1, be1, W2, b2, g2, be2, W3, b3,
           Wc, bc):
  N, _ = x.shape
  E = edge_index.shape[1]
  grp = _NC * _NS * _LANE * 8
  epad = ((E + grp - 1) // grp) * grp
  idt = edge_index.dtype
  src = jnp.concatenate([edge_index[0], jnp.zeros((epad - E,), idt)])
  dst = jnp.concatenate([edge_index[1], jnp.full((epad - E,), N, idt)])
  src2 = src.reshape(epad // _LANE, _LANE)
  dst2 = dst.reshape(epad // _LANE, _LANE)
  kch = epad // (_NC * _NS * _LANE)
  blk = _NS * _LANE
  nacc = ((N + 1 + blk - 1) // blk) * blk

  deg3 = _deg_kernel(kch, nacc)(dst2).reshape(_NC, nacc, 1)

  f1, f2, f3 = W1.shape[1], W2.shape[1], W3.shape[1]
  h1p = pl.pallas_call(
      _proj1_body,
      out_shape=jax.ShapeDtypeStruct((N, f1), jnp.float32))(x, W1, deg3)
  p1 = _scatter_kernel(kch, nacc, f1)(src2, dst2, h1p)
  h2p = pl.pallas_call(
      _mid_body,
      out_shape=jax.ShapeDtypeStruct((N, f2), jnp.float32))(
          p1, h1p, deg3, b1.reshape(1, -1), g1.reshape(1, -1),
          be1.reshape(1, -1), W2)
  p2 = _scatter_kernel(kch, nacc, f2)(src2, dst2, h2p)
  h3p = pl.pallas_call(
      _mid_body,
      out_shape=jax.ShapeDtypeStruct((N, f3), jnp.float32))(
          p2, h2p, deg3, b2.reshape(1, -1), g2.reshape(1, -1),
          be2.reshape(1, -1), W3)
  p3 = _scatter_kernel(kch, nacc, f3)(src2, dst2, h3p)
  out = pl.pallas_call(
      _final_body,
      out_shape=jax.ShapeDtypeStruct((_G, 1), jnp.float32))(
          p3, h3p, deg3, b3.reshape(1, -1), batch.reshape(1, -1), Wc,
          bc.reshape(1, -1))
  return out


# speedup vs baseline: 2.1905x; 2.1905x over previous
import functools

import jax
import jax.numpy as jnp
from jax import lax
from jax.experimental import pallas as pl
from jax.experimental.pallas import tpu as pltpu
from jax.experimental.pallas import tpu_sc as plsc

_EPS = 1e-5
_G = 64
_NC = 2
_NS = 16
_LANE = 128


def _mesh():
  return plsc.VectorSubcoreMesh(
      core_axis_name="c", subcore_axis_name="s",
      num_cores=_NC, num_subcores=_NS)


_SC_PARAMS = pltpu.CompilerParams(use_tc_tiling_on_sc=False)


def _deg_kernel(kch, nacc):
  @functools.partial(
      pl.kernel,
      out_type=jax.ShapeDtypeStruct((_NC * nacc,), jnp.float32),
      mesh=_mesh(),
      compiler_params=_SC_PARAMS,
      scratch_types=[
          pltpu.VMEM((kch, _LANE), jnp.int32),
          pltpu.VMEM((_LANE,), jnp.float32),
          pltpu.VMEM_SHARED((nacc,), jnp.float32),
      ])
  def dk(dst_hbm, out_hbm, didx, vals, acc):
    c = lax.axis_index("c")
    s = lax.axis_index("s")
    wid = c * _NS + s

    @pl.loop(0, _LANE // 16)
    def _zero_vals(i):
      vals[pl.ds(i * 16, 16)] = jnp.zeros((16,), jnp.float32)

    base = s * (nacc // _NS)

    @pl.loop(0, nacc // _NS // _LANE)
    def _zero_acc(i):
      pltpu.sync_copy(vals, acc.at[pl.ds(base + i * _LANE, _LANE)])

    @pl.loop(0, _LANE // 16)
    def _one_vals(i):
      vals[pl.ds(i * 16, 16)] = jnp.ones((16,), jnp.float32)

    pltpu.sync_copy(dst_hbm.at[pl.ds(wid * kch, kch)], didx)
    plsc.subcore_barrier()

    @pl.loop(0, kch)
    def _edges(j):
      pltpu.sync_copy(vals, acc.at[didx.at[j]], add=True)

    plsc.subcore_barrier()
    rpt = nacc // _NS
    pltpu.sync_copy(acc.at[pl.ds(s * rpt, rpt)],
                    out_hbm.at[pl.ds(c * nacc + s * rpt, rpt)])

  return dk


def _scatter_kernel(kch, nacc, F):
  @functools.partial(
      pl.kernel,
      out_type=jax.ShapeDtypeStruct((_NC, nacc, F), jnp.float32),
      mesh=_mesh(),
      compiler_params=_SC_PARAMS,
      scratch_types=[
          pltpu.VMEM((kch, _LANE), jnp.int32),
          pltpu.VMEM((kch, _LANE), jnp.int32),
          pltpu.VMEM((2, _LANE, F), jnp.float32),
          pltpu.VMEM_SHARED((nacc, F), jnp.float32),
          pltpu.SemaphoreType.DMA,
          pltpu.SemaphoreType.DMA,
      ])
  def sk(src_hbm, dst_hbm, hp_hbm, out_hbm, sidx, didx, rows, acc, sem0,
         sem1):
    c = lax.axis_index("c")
    s = lax.axis_index("s")
    wid = c * _NS + s
    sems = (sem0, sem1)

    @pl.loop(0, _LANE)
    def _zero_rows(i):
      for j in range(F // 16):
        rows[0, i, pl.ds(j * 16, 16)] = jnp.zeros((16,), jnp.float32)

    base = s * (nacc // _NS)

    @pl.loop(0, nacc // _NS // _LANE)
    def _zero_acc(i):
      pltpu.sync_copy(rows.at[0], acc.at[pl.ds(base + i * _LANE, _LANE)])

    pltpu.sync_copy(src_hbm.at[pl.ds(wid * kch, kch)], sidx)
    pltpu.sync_copy(dst_hbm.at[pl.ds(wid * kch, kch)], didx)
    plsc.subcore_barrier()

    def _fire(j, b):
      pltpu.async_copy(hp_hbm.at[sidx.at[j]], rows.at[b], sems[b])

    def _drain_scatter(j, b):
      pltpu.make_async_copy(hp_hbm.at[sidx.at[j]], rows.at[b],
                            sems[b]).wait()
      pltpu.sync_copy(rows.at[b], acc.at[didx.at[j]], add=True)

    _fire(0, 0)

    @pl.loop(0, kch // 2)
    def _edges(p):
      j = p * 2
      _fire(j + 1, 1)
      _drain_scatter(j, 0)

      @pl.when(j + 2 < kch)
      def _next():
        _fire(j + 2, 0)

      _drain_scatter(j + 1, 1)

    plsc.subcore_barrier()
    rpt = nacc // _NS
    ob = s * rpt
    pltpu.sync_copy(acc.at[pl.ds(ob, rpt)], out_hbm.at[c, pl.ds(ob, rpt)])

  return sk


def _proj1_body(x_ref, w_ref, deg_ref, out_ref):
  n = x_ref.shape[0]
  dinv = lax.rsqrt(deg_ref[0, :n] + deg_ref[1, :n] + 1.0)
  h = jnp.dot(x_ref[...], w_ref[...], preferred_element_type=jnp.float32)
  out_ref[...] = h * dinv


def _mid_body(p_ref, hp_ref, deg_ref, b_ref, g_ref, be_ref, w_ref, out_ref):
  n = hp_ref.shape[0]
  dinv = lax.rsqrt(deg_ref[0, :n] + deg_ref[1, :n] + 1.0)
  tmp = p_ref[0, :n] + p_ref[1, :n] + hp_ref[...]
  h = tmp * dinv + b_ref[...]
  t = jnp.tanh(h)
  mu = jnp.mean(t, axis=0, keepdims=True)
  var = jnp.mean((t - mu) * (t - mu), axis=0, keepdims=True)
  hbn = g_ref[...] * (t - mu) * lax.rsqrt(var + _EPS) + be_ref[...]
  out_ref[...] = jnp.dot(
      hbn, w_ref[...], preferred_element_type=jnp.float32) * dinv


def _final_body(p_ref, hp_ref, deg_ref, b_ref, batch_ref, wc_ref, bc_ref,
                out_ref):
  n = hp_ref.shape[0]
  dinv = lax.rsqrt(deg_ref[0, :n] + deg_ref[1, :n] + 1.0)
  h = (p_ref[0, :n] + p_ref[1, :n] + hp_ref[...]) * dinv + b_ref[...]
  t = jnp.tanh(h)
  gid = lax.broadcasted_iota(jnp.int32, (_G, n), 0)
  onehot = (gid == batch_ref[...]).astype(jnp.float32)
  sums = jnp.dot(onehot, t, preferred_element_type=jnp.float32)
  counts = jnp.sum(onehot, axis=1, keepdims=True)
  pooled = sums / jnp.maximum(counts, 1.0)
  out_ref[...] = jnp.dot(
      pooled, wc_ref[...], preferred_element_type=jnp.float32) + bc_ref[...]


def kernel(x, edge_index, batch, W1, b1, g1, be1, W2, b2, g2, be2, W3, b3,
           Wc, bc):
  N, _ = x.shape
  E = edge_index.shape[1]
  grp = _NC * _NS * _LANE * 8
  epad = ((E + grp - 1) // grp) * grp
  idt = edge_index.dtype
  kch = epad // (_NC * _NS * _LANE)
  blk = _NS * _LANE
  nacc = ((N + 1 + blk - 1) // blk) * blk
  pad = jnp.arange(epad - E, dtype=idt)
  src = jnp.concatenate([edge_index[0], pad % N])
  dst = jnp.concatenate([edge_index[1], N + pad % (nacc - N)])
  src2 = src.reshape(epad // _LANE, _LANE)
  dst2 = dst.reshape(epad // _LANE, _LANE)

  deg3 = _deg_kernel(kch, nacc)(dst2).reshape(_NC, nacc, 1)

  f1, f2, f3 = W1.shape[1], W2.shape[1], W3.shape[1]
  h1p = pl.pallas_call(
      _proj1_body,
      out_shape=jax.ShapeDtypeStruct((N, f1), jnp.float32))(x, W1, deg3)
  p1 = _scatter_kernel(kch, nacc, f1)(src2, dst2, h1p)
  h2p = pl.pallas_call(
      _mid_body,
      out_shape=jax.ShapeDtypeStruct((N, f2), jnp.float32))(
          p1, h1p, deg3, b1.reshape(1, -1), g1.reshape(1, -1),
          be1.reshape(1, -1), W2)
  p2 = _scatter_kernel(kch, nacc, f2)(src2, dst2, h2p)
  h3p = pl.pallas_call(
      _mid_body,
      out_shape=jax.ShapeDtypeStruct((N, f3), jnp.float32))(
          p2, h2p, deg3, b2.reshape(1, -1), g2.reshape(1, -1),
          be2.reshape(1, -1), W3)
  p3 = _scatter_kernel(kch, nacc, f3)(src2, dst2, h3p)
  out = pl.pallas_call(
      _final_body,
      out_shape=jax.ShapeDtypeStruct((_G, 1), jnp.float32))(
          p3, h3p, deg3, b3.reshape(1, -1), batch.reshape(1, -1), Wc,
          bc.reshape(1, -1))
  return out


# speedup vs baseline: 2.4950x; 1.1390x over previous
import functools

import jax
import jax.numpy as jnp
from jax import lax
from jax.experimental import pallas as pl
from jax.experimental.pallas import tpu as pltpu
from jax.experimental.pallas import tpu_sc as plsc

_EPS = 1e-5
_G = 64
_NC = 2
_NS = 16
_LANE = 128
_NBUF = 8
_LOOK = 4


def _mesh():
  return plsc.VectorSubcoreMesh(
      core_axis_name="c", subcore_axis_name="s",
      num_cores=_NC, num_subcores=_NS)


_SC_PARAMS = pltpu.CompilerParams(use_tc_tiling_on_sc=False)


def _deg_kernel(kch, nacc):
  @functools.partial(
      pl.kernel,
      out_type=jax.ShapeDtypeStruct((_NC * nacc,), jnp.float32),
      mesh=_mesh(),
      compiler_params=_SC_PARAMS,
      scratch_types=[
          pltpu.VMEM((kch, _LANE), jnp.int32),
          pltpu.VMEM((_LANE,), jnp.float32),
          pltpu.VMEM_SHARED((nacc,), jnp.float32),
      ])
  def dk(dst_hbm, out_hbm, didx, vals, acc):
    c = lax.axis_index("c")
    s = lax.axis_index("s")
    wid = c * _NS + s

    @pl.loop(0, _LANE // 16)
    def _zero_vals(i):
      vals[pl.ds(i * 16, 16)] = jnp.zeros((16,), jnp.float32)

    base = s * (nacc // _NS)

    @pl.loop(0, nacc // _NS // _LANE)
    def _zero_acc(i):
      pltpu.sync_copy(vals, acc.at[pl.ds(base + i * _LANE, _LANE)])

    @pl.loop(0, _LANE // 16)
    def _one_vals(i):
      vals[pl.ds(i * 16, 16)] = jnp.ones((16,), jnp.float32)

    pltpu.sync_copy(dst_hbm.at[pl.ds(wid * kch, kch)], didx)
    plsc.subcore_barrier()

    @pl.loop(0, kch)
    def _edges(j):
      pltpu.sync_copy(vals, acc.at[didx.at[j]], add=True)

    plsc.subcore_barrier()
    rpt = nacc // _NS
    pltpu.sync_copy(acc.at[pl.ds(s * rpt, rpt)],
                    out_hbm.at[pl.ds(c * nacc + s * rpt, rpt)])

  return dk


def _scatter_kernel(kch, nacc, F):
  @functools.partial(
      pl.kernel,
      out_type=jax.ShapeDtypeStruct((_NC, nacc, F), jnp.float32),
      mesh=_mesh(),
      compiler_params=_SC_PARAMS,
      scratch_types=[
          pltpu.VMEM((kch, _LANE), jnp.int32),
          pltpu.VMEM((kch, _LANE), jnp.int32),
          pltpu.VMEM((_NBUF, _LANE, F), jnp.float32),
          pltpu.VMEM_SHARED((nacc, F), jnp.float32),
          [pltpu.SemaphoreType.DMA] * _NBUF,
          [pltpu.SemaphoreType.DMA] * _NBUF,
      ])
  def sk(src_hbm, dst_hbm, hp_hbm, out_hbm, sidx, didx, rows, acc, gsems,
         ssems):
    c = lax.axis_index("c")
    s = lax.axis_index("s")
    wid = c * _NS + s

    @pl.loop(0, _LANE)
    def _zero_rows(i):
      for j in range(F // 16):
        rows[0, i, pl.ds(j * 16, 16)] = jnp.zeros((16,), jnp.float32)

    base = s * (nacc // _NS)

    @pl.loop(0, nacc // _NS // _LANE)
    def _zero_acc(i):
      pltpu.sync_copy(rows.at[0], acc.at[pl.ds(base + i * _LANE, _LANE)])

    pltpu.sync_copy(src_hbm.at[pl.ds(wid * kch, kch)], sidx)
    pltpu.sync_copy(dst_hbm.at[pl.ds(wid * kch, kch)], didx)
    plsc.subcore_barrier()

    def _fire_gather(j, b):
      pltpu.async_copy(hp_hbm.at[sidx.at[j]], rows.at[b], gsems[b])

    def _wait_gather(j, b):
      pltpu.make_async_copy(hp_hbm.at[sidx.at[j]], rows.at[b],
                            gsems[b]).wait()

    def _fire_scatter(j, b):
      pltpu.async_copy(rows.at[b], acc.at[didx.at[j]], ssems[b], add=True)

    def _wait_scatter(j, b):
      pltpu.make_async_copy(rows.at[b], acc.at[didx.at[j]],
                            ssems[b]).wait()

    for r in range(_LOOK):
      _fire_gather(r, r)

    @pl.loop(0, kch // _NBUF)
    def _edges(p):
      for r in range(_NBUF):
        j = p * _NBUF + r
        _wait_gather(j, r)
        _fire_scatter(j, r)
        jn = j + _LOOK
        bn = (r + _LOOK) % _NBUF

        @pl.when(jn < kch)
        def _prefetch():
          @pl.when(j >= _LOOK)
          def _reclaim():
            _wait_scatter(jn - _NBUF, bn)

          _fire_gather(jn, bn)

    for r in range(_NBUF):
      _wait_scatter(kch - _NBUF + r, r)

    plsc.subcore_barrier()
    rpt = nacc // _NS
    ob = s * rpt
    pltpu.sync_copy(acc.at[pl.ds(ob, rpt)], out_hbm.at[c, pl.ds(ob, rpt)])

  return sk


def _proj1_body(x_ref, w_ref, deg_ref, out_ref):
  n = x_ref.shape[0]
  dinv = lax.rsqrt(deg_ref[0, :n] + deg_ref[1, :n] + 1.0)
  h = jnp.dot(x_ref[...], w_ref[...], preferred_element_type=jnp.float32)
  out_ref[...] = h * dinv


def _mid_body(p_ref, hp_ref, deg_ref, b_ref, g_ref, be_ref, w_ref, out_ref):
  n = hp_ref.shape[0]
  dinv = lax.rsqrt(deg_ref[0, :n] + deg_ref[1, :n] + 1.0)
  tmp = p_ref[0, :n] + p_ref[1, :n] + hp_ref[...]
  h = tmp * dinv + b_ref[...]
  t = jnp.tanh(h)
  mu = jnp.mean(t, axis=0, keepdims=True)
  var = jnp.mean((t - mu) * (t - mu), axis=0, keepdims=True)
  hbn = g_ref[...] * (t - mu) * lax.rsqrt(var + _EPS) + be_ref[...]
  out_ref[...] = jnp.dot(
      hbn, w_ref[...], preferred_element_type=jnp.float32) * dinv


def _final_body(p_ref, hp_ref, deg_ref, b_ref, batch_ref, wc_ref, bc_ref,
                out_ref):
  n = hp_ref.shape[0]
  dinv = lax.rsqrt(deg_ref[0, :n] + deg_ref[1, :n] + 1.0)
  h = (p_ref[0, :n] + p_ref[1, :n] + hp_ref[...]) * dinv + b_ref[...]
  t = jnp.tanh(h)
  gid = lax.broadcasted_iota(jnp.int32, (_G, n), 0)
  onehot = (gid == batch_ref[...]).astype(jnp.float32)
  sums = jnp.dot(onehot, t, preferred_element_type=jnp.float32)
  counts = jnp.sum(onehot, axis=1, keepdims=True)
  pooled = sums / jnp.maximum(counts, 1.0)
  out_ref[...] = jnp.dot(
      pooled, wc_ref[...], preferred_element_type=jnp.float32) + bc_ref[...]


def kernel(x, edge_index, batch, W1, b1, g1, be1, W2, b2, g2, be2, W3, b3,
           Wc, bc):
  N, _ = x.shape
  E = edge_index.shape[1]
  grp = _NC * _NS * _LANE * 8
  epad = ((E + grp - 1) // grp) * grp
  idt = edge_index.dtype
  kch = epad // (_NC * _NS * _LANE)
  blk = _NS * _LANE
  nacc = ((N + 1 + blk - 1) // blk) * blk
  pad = jnp.arange(epad - E, dtype=idt)
  src = jnp.concatenate([edge_index[0], pad % N])
  dst = jnp.concatenate([edge_index[1], N + pad % (nacc - N)])
  src2 = src.reshape(epad // _LANE, _LANE)
  dst2 = dst.reshape(epad // _LANE, _LANE)

  deg3 = _deg_kernel(kch, nacc)(dst2).reshape(_NC, nacc, 1)

  f1, f2, f3 = W1.shape[1], W2.shape[1], W3.shape[1]
  h1p = pl.pallas_call(
      _proj1_body,
      out_shape=jax.ShapeDtypeStruct((N, f1), jnp.float32))(x, W1, deg3)
  p1 = _scatter_kernel(kch, nacc, f1)(src2, dst2, h1p)
  h2p = pl.pallas_call(
      _mid_body,
      out_shape=jax.ShapeDtypeStruct((N, f2), jnp.float32))(
          p1, h1p, deg3, b1.reshape(1, -1), g1.reshape(1, -1),
          be1.reshape(1, -1), W2)
  p2 = _scatter_kernel(kch, nacc, f2)(src2, dst2, h2p)
  h3p = pl.pallas_call(
      _mid_body,
      out_shape=jax.ShapeDtypeStruct((N, f3), jnp.float32))(
          p2, h2p, deg3, b2.reshape(1, -1), g2.reshape(1, -1),
          be2.reshape(1, -1), W3)
  p3 = _scatter_kernel(kch, nacc, f3)(src2, dst2, h3p)
  out = pl.pallas_call(
      _final_body,
      out_shape=jax.ShapeDtypeStruct((_G, 1), jnp.float32))(
          p3, h3p, deg3, b3.reshape(1, -1), batch.reshape(1, -1), Wc,
          bc.reshape(1, -1))
  return out


# speedup vs baseline: 2.7000x; 1.0822x over previous
import functools

import jax
import jax.numpy as jnp
from jax import lax
from jax.experimental import pallas as pl
from jax.experimental.pallas import tpu as pltpu
from jax.experimental.pallas import tpu_sc as plsc

_EPS = 1e-5
_G = 64
_NC = 2
_NS = 16
_LANE = 128
_NBUF = 8
_LOOK = 4


def _mesh():
  return plsc.VectorSubcoreMesh(
      core_axis_name="c", subcore_axis_name="s",
      num_cores=_NC, num_subcores=_NS)


_SC_PARAMS = pltpu.CompilerParams(use_tc_tiling_on_sc=False,
                                  needs_layout_passes=False)


def _deg_kernel(kch, nacc):
  @functools.partial(
      pl.kernel,
      out_type=jax.ShapeDtypeStruct((_NC, nacc, 1), jnp.float32),
      mesh=_mesh(),
      compiler_params=_SC_PARAMS,
      scratch_types=[
          pltpu.VMEM((kch, _LANE), jnp.int32),
          pltpu.VMEM((_LANE, 1), jnp.float32),
          pltpu.VMEM_SHARED((nacc, 1), jnp.float32),
      ])
  def dk(dst_hbm, out_hbm, didx, vals, acc):
    c = lax.axis_index("c")
    s = lax.axis_index("s")
    wid = c * _NS + s

    def _fill_vals(v):
      @pl.loop(0, _LANE // 16)
      def _f(i):
        rows = lax.iota(jnp.int32, 16) + i * 16
        cols = jnp.zeros((16,), jnp.int32)
        plsc.store_scatter(vals, [rows, cols],
                           jnp.full((16,), v, jnp.float32))

    _fill_vals(0.0)

    base = s * (nacc // _NS)

    @pl.loop(0, nacc // _NS // _LANE)
    def _zero_acc(i):
      pltpu.sync_copy(vals, acc.at[pl.ds(base + i * _LANE, _LANE)])

    _fill_vals(1.0)

    pltpu.sync_copy(dst_hbm.at[pl.ds(wid * kch, kch)], didx)
    plsc.subcore_barrier()

    @pl.loop(0, kch)
    def _edges(j):
      pltpu.sync_copy(vals, acc.at[didx.at[j]], add=True)

    plsc.subcore_barrier()
    rpt = nacc // _NS
    pltpu.sync_copy(acc.at[pl.ds(s * rpt, rpt)],
                    out_hbm.at[c, pl.ds(s * rpt, rpt)])

  return dk


def _scatter_kernel(kch, nacc, F):
  @functools.partial(
      pl.kernel,
      out_type=jax.ShapeDtypeStruct((nacc, _NC * F), jnp.float32),
      mesh=_mesh(),
      compiler_params=_SC_PARAMS,
      scratch_types=[
          pltpu.VMEM((kch, _LANE), jnp.int32),
          pltpu.VMEM((kch, _LANE), jnp.int32),
          pltpu.VMEM((_NBUF, _LANE, F), jnp.float32),
          pltpu.VMEM_SHARED((nacc, F), jnp.float32),
          [pltpu.SemaphoreType.DMA] * _NBUF,
          [pltpu.SemaphoreType.DMA] * _NBUF,
      ])
  def sk(src_hbm, dst_hbm, hp_hbm, out_hbm, sidx, didx, rows, acc, gsems,
         ssems):
    c = lax.axis_index("c")
    s = lax.axis_index("s")
    wid = c * _NS + s

    @pl.loop(0, _LANE)
    def _zero_rows(i):
      for j in range(F // 16):
        rows[0, i, pl.ds(j * 16, 16)] = jnp.zeros((16,), jnp.float32)

    base = s * (nacc // _NS)

    @pl.loop(0, nacc // _NS // _LANE)
    def _zero_acc(i):
      pltpu.sync_copy(rows.at[0], acc.at[pl.ds(base + i * _LANE, _LANE)])

    pltpu.sync_copy(src_hbm.at[pl.ds(wid * kch, kch)], sidx)
    pltpu.sync_copy(dst_hbm.at[pl.ds(wid * kch, kch)], didx)
    plsc.subcore_barrier()

    def _fire_gather(j, b):
      pltpu.async_copy(hp_hbm.at[sidx.at[j]], rows.at[b], gsems[b])

    def _wait_gather(j, b):
      pltpu.make_async_copy(hp_hbm.at[sidx.at[j]], rows.at[b],
                            gsems[b]).wait()

    def _fire_scatter(j, b):
      pltpu.async_copy(rows.at[b], acc.at[didx.at[j]], ssems[b], add=True)

    def _wait_scatter(j, b):
      pltpu.make_async_copy(rows.at[b], acc.at[didx.at[j]],
                            ssems[b]).wait()

    for r in range(_LOOK):
      _fire_gather(r, r)

    @pl.loop(0, kch // _NBUF)
    def _edges(p):
      for r in range(_NBUF):
        j = p * _NBUF + r
        _wait_gather(j, r)
        _fire_scatter(j, r)
        jn = j + _LOOK
        bn = (r + _LOOK) % _NBUF

        @pl.when(jn < kch)
        def _prefetch():
          @pl.when(j >= _LOOK)
          def _reclaim():
            _wait_scatter(jn - _NBUF, bn)

          _fire_gather(jn, bn)

    for r in range(_NBUF):
      _wait_scatter(kch - _NBUF + r, r)

    plsc.subcore_barrier()
    rpt = nacc // _NS
    ob = s * rpt
    pltpu.sync_copy(acc.at[pl.ds(ob, rpt)],
                    out_hbm.at[pl.ds(ob, rpt), pl.ds(c * F, F)])

  return sk


def _proj1_body(x_ref, w_ref, deg_ref, out_ref):
  n = x_ref.shape[0]
  dinv = lax.rsqrt(deg_ref[0, :n] + deg_ref[1, :n] + 1.0)
  h = jnp.dot(x_ref[...], w_ref[...], preferred_element_type=jnp.float32)
  out_ref[...] = h * dinv


def _mid_body(p_ref, hp_ref, deg_ref, b_ref, g_ref, be_ref, w_ref, out_ref):
  n, f = hp_ref.shape
  dinv = lax.rsqrt(deg_ref[0, :n] + deg_ref[1, :n] + 1.0)
  tmp = p_ref[:n, :f] + p_ref[:n, f:] + hp_ref[...]
  h = tmp * dinv + b_ref[...]
  t = jnp.tanh(h)
  mu = jnp.mean(t, axis=0, keepdims=True)
  var = jnp.mean((t - mu) * (t - mu), axis=0, keepdims=True)
  hbn = g_ref[...] * (t - mu) * lax.rsqrt(var + _EPS) + be_ref[...]
  out_ref[...] = jnp.dot(
      hbn, w_ref[...], preferred_element_type=jnp.float32) * dinv


def _final_body(p_ref, hp_ref, deg_ref, b_ref, batch_ref, wc_ref, bc_ref,
                out_ref):
  n, f = hp_ref.shape
  dinv = lax.rsqrt(deg_ref[0, :n] + deg_ref[1, :n] + 1.0)
  h = (p_ref[:n, :f] + p_ref[:n, f:] + hp_ref[...]) * dinv + b_ref[...]
  t = jnp.tanh(h)
  gid = lax.broadcasted_iota(jnp.int32, (_G, n), 0)
  onehot = (gid == batch_ref[...]).astype(jnp.float32)
  sums = jnp.dot(onehot, t, preferred_element_type=jnp.float32)
  counts = jnp.sum(onehot, axis=1, keepdims=True)
  pooled = sums / jnp.maximum(counts, 1.0)
  out_ref[...] = jnp.dot(
      pooled, wc_ref[...], preferred_element_type=jnp.float32) + bc_ref[...]


def kernel(x, edge_index, batch, W1, b1, g1, be1, W2, b2, g2, be2, W3, b3,
           Wc, bc):
  N, _ = x.shape
  E = edge_index.shape[1]
  grp = _NC * _NS * _LANE * 8
  epad = ((E + grp - 1) // grp) * grp
  idt = edge_index.dtype
  kch = epad // (_NC * _NS * _LANE)
  blk = _NS * _LANE
  nacc = ((N + 1 + blk - 1) // blk) * blk
  pad = jnp.arange(epad - E, dtype=idt)
  src = jnp.concatenate([edge_index[0], pad % N])
  dst = jnp.concatenate([edge_index[1], N + pad % (nacc - N)])
  src2 = src.reshape(epad // _LANE, _LANE)
  dst2 = dst.reshape(epad // _LANE, _LANE)

  deg3 = _deg_kernel(kch, nacc)(dst2)

  f1, f2, f3 = W1.shape[1], W2.shape[1], W3.shape[1]
  h1p = pl.pallas_call(
      _proj1_body,
      out_shape=jax.ShapeDtypeStruct((N, f1), jnp.float32))(x, W1, deg3)
  p1 = _scatter_kernel(kch, nacc, f1)(src2, dst2, h1p)
  h2p = pl.pallas_call(
      _mid_body,
      out_shape=jax.ShapeDtypeStruct((N, f2), jnp.float32))(
          p1, h1p, deg3, b1.reshape(1, -1), g1.reshape(1, -1),
          be1.reshape(1, -1), W2)
  p2 = _scatter_kernel(kch, nacc, f2)(src2, dst2, h2p)
  h3p = pl.pallas_call(
      _mid_body,
      out_shape=jax.ShapeDtypeStruct((N, f3), jnp.float32))(
          p2, h2p, deg3, b2.reshape(1, -1), g2.reshape(1, -1),
          be2.reshape(1, -1), W3)
  p3 = _scatter_kernel(kch, nacc, f3)(src2, dst2, h3p)
  out = pl.pallas_call(
      _final_body,
      out_shape=jax.ShapeDtypeStruct((_G, 1), jnp.float32))(
          p3, h3p, deg3, b3.reshape(1, -1), batch.reshape(1, -1), Wc,
          bc.reshape(1, -1))
  return out
